# baseline (speedup 1.0000x reference)
import jax
import jax.numpy as jnp
from jax.experimental import pallas as pl
from jax.experimental.pallas import tpu as pltpu

_H0 = 8


def _fwd_kernel(adj_ref, x_ref, emb_ref, w0_ref, asrc0_ref, adst0_ref, b0_ref,
                w1_ref, asrc1_ref, adst1_ref, b1_ref, out_ref, w0w_ref):
    b = pl.program_id(0)
    adj = adj_ref[0]
    n = adj.shape[0]
    f0 = w0_ref.shape[2]
    f1 = out_ref.shape[1]
    fx = x_ref.shape[2]
    adjf = adj.astype(jnp.float32)

    @pl.when(b == 0)
    def _():
        for h in range(_H0):
            w0w_ref[:, h * f0:(h + 1) * f0] = w0_ref[h]

    hp_all = (jnp.dot(x_ref[0], w0w_ref[:fx, :], preferred_element_type=jnp.float32)
              + jnp.dot(emb_ref[0], w0w_ref[fx:, :], preferred_element_type=jnp.float32))

    w1 = w1_ref[0]
    h1 = jnp.zeros((n, f1), jnp.float32)
    for h in range(_H0):
        hp = hp_all[:, h * f0:(h + 1) * f0]
        asrc_v = asrc0_ref[h].reshape(1, f0)
        adst_v = adst0_ref[h].reshape(1, f0)
        asrc = jnp.sum(hp * asrc_v, axis=1, keepdims=True)
        adst = jnp.sum(hp * adst_v, axis=1, keepdims=True)
        mdst = jnp.max(adst)
        sm = asrc + mdst
        m = jnp.maximum(sm, 0.2 * sm)
        l = asrc + adst.reshape(1, n)
        l = jnp.maximum(l, 0.2 * l)
        e = jnp.exp(l - m) * adjf
        s = jnp.sum(e, axis=1, keepdims=True)
        o = jnp.dot(e, hp, preferred_element_type=jnp.float32) / s + b0_ref[...][None, :]
        col = jnp.where(o > 0, o, jnp.exp(jnp.minimum(o, 0.0)) - 1.0)
        h1 = h1 + jnp.dot(col, w1[h * f0:(h + 1) * f0, :],
                          preferred_element_type=jnp.float32)

    asrc1_v = asrc1_ref[0].reshape(1, f1)
    adst1_v = adst1_ref[0].reshape(1, f1)
    adst1 = jnp.sum(h1 * adst1_v, axis=1, keepdims=True)
    asrc1 = jnp.sum(h1[n - 1:n, :] * asrc1_v, axis=1, keepdims=True)
    row = asrc1 + adst1.reshape(1, n)
    row = jnp.maximum(row, 0.2 * row)
    md1 = jnp.maximum(asrc1 + jnp.max(adst1), 0.2 * (asrc1 + jnp.max(adst1)))
    e2 = jnp.exp(row - md1) * adjf[n - 1:n, :]
    s2 = jnp.sum(e2, axis=1, keepdims=True)
    o2 = jnp.dot(e2, h1, preferred_element_type=jnp.float32) / s2 + b1_ref[...][None, :]
    m3 = jnp.max(o2, axis=1, keepdims=True)
    l3 = o2 - m3
    out_ref[pl.ds(b, 1), :] = l3 - jnp.log(jnp.sum(jnp.exp(l3), axis=1, keepdims=True))


def kernel(adj, x, normalized_embedding, w0, a_src0, a_dst0, b0,
           w1, a_src1, a_dst1, b1):
    bs, n = adj.shape[:2]
    f_x = x.shape[2]
    f_emb = normalized_embedding.shape[2]
    f_in = f_x + f_emb
    f_out0 = w0.shape[2]
    f_out1 = w1.shape[2]

    grid = (bs,)
    batch3 = lambda b: (b, 0, 0)
    const3 = lambda b: (0, 0, 0)
    return pl.pallas_call(
        _fwd_kernel,
        grid=grid,
        in_specs=[
            pl.BlockSpec((1, n, n), batch3),
            pl.BlockSpec((1, n, f_x), batch3),
            pl.BlockSpec((1, n, f_emb), batch3),
            pl.BlockSpec(w0.shape, const3),
            pl.BlockSpec(a_src0.shape, const3),
            pl.BlockSpec(a_dst0.shape, const3),
            pl.BlockSpec(b0.shape, lambda b: (0,)),
            pl.BlockSpec(w1.shape, const3),
            pl.BlockSpec(a_src1.shape, const3),
            pl.BlockSpec(a_dst1.shape, const3),
            pl.BlockSpec(b1.shape, lambda b: (0,)),
        ],
        out_specs=pl.BlockSpec((bs, f_out1), lambda b: (0, 0)),
        out_shape=jax.ShapeDtypeStruct((bs, f_out1), jnp.float32),
        scratch_shapes=[pltpu.VMEM((f_in, _H0 * f_out0), jnp.float32)],
        compiler_params=pltpu.CompilerParams(
            dimension_semantics=("arbitrary",)),
    )(adj, x, normalized_embedding, w0, a_src0, a_dst0, b0,
      w1, a_src1, a_dst1, b1)


# speedup vs baseline: 1.1601x; 1.1601x over previous
import jax
import jax.numpy as jnp
from jax.experimental import pallas as pl
from jax.experimental.pallas import tpu as pltpu

_H0 = 8


def _fwd_kernel(adj_ref, x_ref, emb_ref, w0_ref,
                asrc0_ref, adst0_ref, b0_ref,
                w1_ref, asrc1_ref, adst1_ref, b1_ref, out_ref, w0w_ref):
    adj = adj_ref[0]
    n = adj.shape[0]
    f0 = b0_ref.shape[1]
    f1 = out_ref.shape[2]
    fx = x_ref.shape[2]
    adjf = adj.astype(jnp.float32)

    @pl.when(pl.program_id(0) == 0)
    def _():
        for h in range(_H0):
            w0w_ref[:, h * f0:(h + 1) * f0] = w0_ref[h]

    hp_all = (jnp.dot(x_ref[0], w0w_ref[:fx, :], preferred_element_type=jnp.float32)
              + jnp.dot(emb_ref[0], w0w_ref[fx:, :], preferred_element_type=jnp.float32))

    h1 = jnp.zeros((n, f1), jnp.float32)
    for h in range(_H0):
        hp = hp_all[:, h * f0:(h + 1) * f0]
        asrc = jnp.sum(hp * asrc0_ref[h][None, :], axis=1, keepdims=True)
        adst = jnp.sum(hp * adst0_ref[h][None, :], axis=1, keepdims=True)
        mdst = jnp.max(adst)
        sm = asrc + mdst
        m = jnp.maximum(sm, 0.2 * sm)
        l = asrc + adst.reshape(1, n)
        l = jnp.maximum(l, 0.2 * l)
        e = jnp.exp(l - m) * adjf
        s = jnp.sum(e, axis=1, keepdims=True)
        o = jnp.dot(e, hp, preferred_element_type=jnp.float32) / s + b0_ref[...]
        col = jnp.where(o > 0, o, jnp.exp(jnp.minimum(o, 0.0)) - 1.0)
        h1 = h1 + jnp.dot(col, w1_ref[h], preferred_element_type=jnp.float32)

    adst1 = jnp.sum(h1 * adst1_ref[...], axis=1, keepdims=True)
    asrc1 = jnp.sum(h1[n - 1:n, :] * asrc1_ref[...], axis=1, keepdims=True)
    row = asrc1 + adst1.reshape(1, n)
    row = jnp.maximum(row, 0.2 * row)
    md1 = jnp.maximum(asrc1 + jnp.max(adst1), 0.2 * (asrc1 + jnp.max(adst1)))
    e2 = jnp.exp(row - md1) * adjf[n - 1:n, :]
    s2 = jnp.sum(e2, axis=1, keepdims=True)
    o2 = jnp.dot(e2, h1, preferred_element_type=jnp.float32) / s2 + b1_ref[...]
    m3 = jnp.max(o2, axis=1, keepdims=True)
    l3 = o2 - m3
    out_ref[0] = l3 - jnp.log(jnp.sum(jnp.exp(l3), axis=1, keepdims=True))


def kernel(adj, x, normalized_embedding, w0, a_src0, a_dst0, b0,
           w1, a_src1, a_dst1, b1):
    bs, n = adj.shape[:2]
    f_x = x.shape[2]
    f_emb = normalized_embedding.shape[2]
    f_in = f_x + f_emb
    f_out0 = w0.shape[2]
    f_out1 = w1.shape[2]

    w1_heads = w1.reshape(_H0, f_out0, f_out1)

    grid = (bs,)
    batch3 = lambda b: (b, 0, 0)
    const2 = lambda b: (0, 0)
    const3 = lambda b: (0, 0, 0)
    out = pl.pallas_call(
        _fwd_kernel,
        grid=grid,
        in_specs=[
            pl.BlockSpec((1, n, n), batch3),
            pl.BlockSpec((1, n, f_x), batch3),
            pl.BlockSpec((1, n, f_emb), batch3),
            pl.BlockSpec(w0.shape, const3),
            pl.BlockSpec((_H0, f_out0), const2),
            pl.BlockSpec((_H0, f_out0), const2),
            pl.BlockSpec((1, f_out0), const2),
            pl.BlockSpec((_H0, f_out0, f_out1), const3),
            pl.BlockSpec((1, f_out1), const2),
            pl.BlockSpec((1, f_out1), const2),
            pl.BlockSpec((1, f_out1), const2),
        ],
        out_specs=pl.BlockSpec((1, 1, f_out1), lambda b: (b, 0, 0)),
        out_shape=jax.ShapeDtypeStruct((bs, 1, f_out1), jnp.float32),
        scratch_shapes=[pltpu.VMEM((f_in, _H0 * f_out0), jnp.float32)],
        compiler_params=pltpu.CompilerParams(
            dimension_semantics=("arbitrary",)),
    )(
        adj, x, normalized_embedding, w0,
        a_src0.reshape(_H0, f_out0), a_dst0.reshape(_H0, f_out0),
        b0.reshape(1, f_out0),
        w1_heads,
        a_src1.reshape(1, f_out1), a_dst1.reshape(1, f_out1),
        b1.reshape(1, f_out1),
    )
    return out.reshape(bs, f_out1)
